# baseline (speedup 1.0000x reference)
import functools

import jax
import jax.numpy as jnp
from jax import lax
from jax.experimental import pallas as pl
from jax.experimental.pallas import tpu as pltpu
from jax.experimental.pallas import tpu_sc as plsc

SUBVERT_NUM = 100000
EMB_DIM = 64
NUM_FILTERS = 128
BATCH = 16384

_info = plsc.get_sparse_core_info()
_NC, _NS = _info.num_cores, _info.num_subcores
_NW = _NC * _NS
_B_PER_W = BATCH // _NW
_CHUNK = 128
_NCHUNK = _B_PER_W // _CHUNK


def _gather_sc(idx3, table):
    @functools.partial(
        pl.kernel,
        mesh=plsc.VectorSubcoreMesh(core_axis_name="c", subcore_axis_name="s"),
        out_type=jax.ShapeDtypeStruct((BATCH, EMB_DIM), jnp.float32),
        scratch_types=[
            pltpu.VMEM((_NCHUNK, _CHUNK), jnp.int32),
            pltpu.VMEM((_B_PER_W, EMB_DIM), jnp.float32),
            pltpu.SemaphoreType.DMA,
        ],
        compiler_params=pltpu.CompilerParams(use_tc_tiling_on_sc=False),
    )
    def k(idx_hbm, table_hbm, out_hbm, idx_v, rows_v, sem):
        wid = lax.axis_index("s") * _NC + lax.axis_index("c")
        base = wid * _B_PER_W
        pltpu.sync_copy(idx_hbm.at[wid], idx_v)
        copies = []
        for j in range(_NCHUNK):
            copies.append(
                pltpu.async_copy(
                    table_hbm.at[idx_v.at[j]],
                    rows_v.at[pl.ds(j * _CHUNK, _CHUNK)],
                    sem,
                )
            )
        for c in copies:
            c.wait()
        pltpu.sync_copy(rows_v, out_hbm.at[pl.ds(base, _B_PER_W)])

    return k(idx3, table)


_TC_BLOCK = 2048


def _proj_body(x_ref, wt_ref, b_ref, o_ref):
    o_ref[...] = (
        jnp.dot(x_ref[...], wt_ref[...], preferred_element_type=jnp.float32)
        + b_ref[...]
    )


def _project_tc(x, wt, b2):
    grid = BATCH // _TC_BLOCK
    return pl.pallas_call(
        _proj_body,
        grid=(grid,),
        in_specs=[
            pl.BlockSpec((_TC_BLOCK, EMB_DIM), lambda i: (i, 0)),
            pl.BlockSpec((EMB_DIM, NUM_FILTERS), lambda i: (0, 0)),
            pl.BlockSpec((1, NUM_FILTERS), lambda i: (0, 0)),
        ],
        out_specs=pl.BlockSpec((_TC_BLOCK, NUM_FILTERS), lambda i: (i, 0)),
        out_shape=jax.ShapeDtypeStruct((BATCH, NUM_FILTERS), jnp.float32),
    )(x, wt, b2)


def kernel(input_subvert, table, W, b):
    idx3 = input_subvert.astype(jnp.int32).reshape(_NW, _NCHUNK, _CHUNK)
    gathered = _gather_sc(idx3, table)
    return _project_tc(gathered, W.T, b.reshape(1, NUM_FILTERS))


# speedup vs baseline: 1.4517x; 1.4517x over previous
import functools

import jax
import jax.numpy as jnp
from jax import lax
from jax.experimental import pallas as pl
from jax.experimental.pallas import tpu as pltpu
from jax.experimental.pallas import tpu_sc as plsc

SUBVERT_NUM = 100000
EMB_DIM = 64
NUM_FILTERS = 128
BATCH = 16384

_info = plsc.get_sparse_core_info()
_NC, _NS = _info.num_cores, _info.num_subcores
_NW = _NC * _NS
_B_PER_W = BATCH // _NW


def _gather_sc(idx2, table):
    @functools.partial(
        pl.kernel,
        mesh=plsc.VectorSubcoreMesh(core_axis_name="c", subcore_axis_name="s"),
        out_type=jax.ShapeDtypeStruct((BATCH, EMB_DIM), jnp.float32),
        scratch_types=[
            pltpu.VMEM((_B_PER_W,), jnp.int32),
            pltpu.VMEM((_B_PER_W, EMB_DIM), jnp.float32),
            pltpu.SemaphoreType.DMA,
            pltpu.SemaphoreType.DMA,
        ],
    )
    def k(idx_hbm, table_hbm, out_hbm, idx_v, rows_v, sem_i, sem):
        wid = lax.axis_index("s") * _NC + lax.axis_index("c")
        base = wid * _B_PER_W
        pltpu.async_copy(idx_hbm.at[wid], idx_v, sem_i).wait()

        def body(j, _):
            vbase = j * 16
            idx_vec = idx_v[pl.ds(vbase, 16)]
            for t in range(16):
                r = idx_vec[t]
                pltpu.async_copy(
                    table_hbm.at[pl.ds(r, 1)], rows_v.at[pl.ds(vbase + t, 1)], sem
                )
            return 0

        lax.fori_loop(0, _B_PER_W // 16, body, 0)

        pltpu.make_async_copy(
            table_hbm.at[pl.ds(0, _B_PER_W)], rows_v, sem
        ).wait()
        pltpu.sync_copy(rows_v, out_hbm.at[pl.ds(base, _B_PER_W)])

    return k(idx2, table)


def _proj_body(x_ref, w_ref, b_ref, o_ref):
    o_ref[...] = (
        lax.dot_general(
            x_ref[...], w_ref[...],
            dimension_numbers=(((1,), (1,)), ((), ())),
            preferred_element_type=jnp.float32,
        )
        + b_ref[...]
    )


_TC_BLOCK = 2048


def _project_tc(x, W, b2):
    grid = BATCH // _TC_BLOCK
    return pl.pallas_call(
        _proj_body,
        grid=(grid,),
        in_specs=[
            pl.BlockSpec((_TC_BLOCK, EMB_DIM), lambda i: (i, 0)),
            pl.BlockSpec((NUM_FILTERS, EMB_DIM), lambda i: (0, 0)),
            pl.BlockSpec((1, NUM_FILTERS), lambda i: (0, 0)),
        ],
        out_specs=pl.BlockSpec((_TC_BLOCK, NUM_FILTERS), lambda i: (i, 0)),
        out_shape=jax.ShapeDtypeStruct((BATCH, NUM_FILTERS), jnp.float32),
    )(x, W, b2)


def kernel(input_subvert, table, W, b):
    idx2 = input_subvert.astype(jnp.int32).reshape(_NW, _B_PER_W)
    gathered = _gather_sc(idx2, table)
    return _project_tc(gathered, W, b.reshape(1, NUM_FILTERS))
